# Initial kernel scaffold; baseline (speedup 1.0000x reference)
import jax
import jax.numpy as jnp
from jax.experimental import pallas as pl


def kernel(source_pc, target_pc):
    raise NotImplementedError("write your pallas kernel here")


# speedup vs baseline: 21.5912x; 21.5912x over previous
import jax
import jax.numpy as jnp
from jax.experimental import pallas as pl

_K = 3
_QBLK = 512


def _knn_kernel(q_ref, rt_ref, sum_ref, cnt_ref):
    q = q_ref[0]
    rt = rt_ref[0]
    nr = rt.shape[1]
    inf = jnp.float32(jnp.inf)

    rvalid = (rt[0:1, :] != 0.0) | (rt[1:2, :] != 0.0) | (rt[2:3, :] != 0.0)

    q2 = jnp.sum(q * q, axis=1, keepdims=True)
    r2 = jnp.sum(rt * rt, axis=0, keepdims=True)
    qr = jax.lax.dot_general(q, rt, (((1,), (0,)), ((), ())),
                             preferred_element_type=jnp.float32)
    d2 = q2 + r2 - 2.0 * qr
    d2 = jnp.where(rvalid, d2, inf)

    iota = jax.lax.broadcasted_iota(jnp.int32, d2.shape, 1)
    s = jnp.zeros((q.shape[0], 1), jnp.float32)
    for _ in range(_K):
        m = jnp.min(d2, axis=1, keepdims=True)
        s = s + jnp.sqrt(jnp.maximum(m, 0.0))
        is_min = d2 == m
        first = jnp.min(jnp.where(is_min, iota, nr), axis=1, keepdims=True)
        d2 = jnp.where(iota == first, inf, d2)

    qvalid = (q[:, 0:1] != 0.0) | (q[:, 1:2] != 0.0) | (q[:, 2:3] != 0.0)
    s = jnp.where(qvalid, s, 0.0)
    sum_ref[...] = jnp.sum(s).reshape(1, 1, 1)
    cnt_ref[...] = jnp.sum(qvalid.astype(jnp.float32)).reshape(1, 1, 1)


def kernel(source_pc, target_pc):
    B, NQ, _ = source_pc.shape
    NR = target_pc.shape[1]
    nqb = NQ // _QBLK
    tt = jnp.transpose(target_pc, (0, 2, 1))
    sums, cnts = pl.pallas_call(
        _knn_kernel,
        grid=(B, nqb),
        in_specs=[
            pl.BlockSpec((1, _QBLK, 3), lambda b, i: (b, i, 0)),
            pl.BlockSpec((1, 3, NR), lambda b, i: (b, 0, 0)),
        ],
        out_specs=[
            pl.BlockSpec((1, 1, 1), lambda b, i: (b * nqb + i, 0, 0)),
            pl.BlockSpec((1, 1, 1), lambda b, i: (b * nqb + i, 0, 0)),
        ],
        out_shape=[
            jax.ShapeDtypeStruct((B * nqb, 1, 1), jnp.float32),
            jax.ShapeDtypeStruct((B * nqb, 1, 1), jnp.float32),
        ],
    )(source_pc, tt)
    total = jnp.sum(sums.reshape(B, nqb), axis=1)
    cnt = jnp.sum(cnts.reshape(B, nqb), axis=1) * _K
    return jnp.mean(total / cnt)


# speedup vs baseline: 24.0950x; 1.1160x over previous
import jax
import jax.numpy as jnp
from jax.experimental import pallas as pl
from jax.experimental.pallas import tpu as pltpu

_K = 3
_QBLK = 512


def _knn_kernel(q_ref, rt_ref, sum_ref, cnt_ref):
    q = q_ref[0]
    rt = rt_ref[0]
    inf = jnp.float32(jnp.inf)

    rvalid = (rt[0:1, :] != 0.0) | (rt[1:2, :] != 0.0) | (rt[2:3, :] != 0.0)
    q2 = jnp.sum(q * q, axis=1, keepdims=True)
    r2 = jnp.sum(rt * rt, axis=0, keepdims=True)
    r2m = jnp.where(rvalid, r2, inf)
    qr = jax.lax.dot_general(q, rt, (((1,), (0,)), ((), ())),
                             preferred_element_type=jnp.float32)
    d2 = q2 + r2m - 2.0 * qr

    m1 = jnp.min(d2, axis=1, keepdims=True)
    e1 = d2 == m1
    c1 = jnp.sum(e1.astype(jnp.int32), axis=1, keepdims=True)
    d2 = jnp.where(e1, inf, d2)
    m2 = jnp.min(d2, axis=1, keepdims=True)
    e2 = d2 == m2
    c2 = jnp.sum(e2.astype(jnp.int32), axis=1, keepdims=True)
    d2 = jnp.where(e2, inf, d2)
    m3 = jnp.min(d2, axis=1, keepdims=True)

    t1 = jnp.minimum(c1, _K)
    t2 = jnp.minimum(c2, _K - t1)
    t3 = _K - t1 - t2
    s1 = jnp.sqrt(jnp.maximum(m1, 0.0))
    s2 = jnp.sqrt(jnp.maximum(m2, 0.0))
    s3 = jnp.sqrt(jnp.maximum(m3, 0.0))
    f32 = jnp.float32
    s = (t1.astype(f32) * s1
         + jnp.where(t2 > 0, t2.astype(f32) * s2, 0.0)
         + jnp.where(t3 > 0, t3.astype(f32) * s3, 0.0))

    qvalid = (q[:, 0:1] != 0.0) | (q[:, 1:2] != 0.0) | (q[:, 2:3] != 0.0)
    s = jnp.where(qvalid, s, 0.0)
    sum_ref[...] = jnp.sum(s).reshape(1, 1, 1)
    cnt_ref[...] = jnp.sum(qvalid.astype(f32)).reshape(1, 1, 1)


def kernel(source_pc, target_pc):
    B, NQ, _ = source_pc.shape
    NR = target_pc.shape[1]
    nqb = NQ // _QBLK
    tt = jnp.transpose(target_pc, (0, 2, 1))
    sums, cnts = pl.pallas_call(
        _knn_kernel,
        grid=(B, nqb),
        in_specs=[
            pl.BlockSpec((1, _QBLK, 3), lambda b, i: (b, i, 0)),
            pl.BlockSpec((1, 3, NR), lambda b, i: (b, 0, 0)),
        ],
        out_specs=[
            pl.BlockSpec((1, 1, 1), lambda b, i: (b * nqb + i, 0, 0)),
            pl.BlockSpec((1, 1, 1), lambda b, i: (b * nqb + i, 0, 0)),
        ],
        out_shape=[
            jax.ShapeDtypeStruct((B * nqb, 1, 1), jnp.float32),
            jax.ShapeDtypeStruct((B * nqb, 1, 1), jnp.float32),
        ],
        compiler_params=pltpu.CompilerParams(
            dimension_semantics=("parallel", "parallel")),
    )(source_pc, tt)
    total = jnp.sum(sums.reshape(B, nqb), axis=1)
    cnt = jnp.sum(cnts.reshape(B, nqb), axis=1) * _K
    return jnp.mean(total / cnt)


# speedup vs baseline: 25.2239x; 1.0469x over previous
import jax
import jax.numpy as jnp
from jax.experimental import pallas as pl
from jax.experimental.pallas import tpu as pltpu

_K = 3
_QBLK = 1024


def _knn_kernel(q_ref, rt_ref, sum_ref, cnt_ref):
    q = q_ref[0]
    rt = rt_ref[0]
    inf = jnp.float32(jnp.inf)

    rvalid = (rt[0:1, :] != 0.0) | (rt[1:2, :] != 0.0) | (rt[2:3, :] != 0.0)
    q2 = jnp.sum(q * q, axis=1, keepdims=True)
    r2 = jnp.sum(rt * rt, axis=0, keepdims=True)
    r2m = jnp.where(rvalid, r2, inf)
    qr = jax.lax.dot_general(q, rt, (((1,), (0,)), ((), ())),
                             preferred_element_type=jnp.float32)
    d2 = q2 + r2m - 2.0 * qr

    m1 = jnp.min(d2, axis=1, keepdims=True)
    e1 = d2 == m1
    c1 = jnp.sum(e1.astype(jnp.int32), axis=1, keepdims=True)
    d2 = jnp.where(e1, inf, d2)
    m2 = jnp.min(d2, axis=1, keepdims=True)
    e2 = d2 == m2
    c2 = jnp.sum(e2.astype(jnp.int32), axis=1, keepdims=True)
    d2 = jnp.where(e2, inf, d2)
    m3 = jnp.min(d2, axis=1, keepdims=True)

    t1 = jnp.minimum(c1, _K)
    t2 = jnp.minimum(c2, _K - t1)
    t3 = _K - t1 - t2
    s1 = jnp.sqrt(jnp.maximum(m1, 0.0))
    s2 = jnp.sqrt(jnp.maximum(m2, 0.0))
    s3 = jnp.sqrt(jnp.maximum(m3, 0.0))
    f32 = jnp.float32
    s = (t1.astype(f32) * s1
         + jnp.where(t2 > 0, t2.astype(f32) * s2, 0.0)
         + jnp.where(t3 > 0, t3.astype(f32) * s3, 0.0))

    qvalid = (q[:, 0:1] != 0.0) | (q[:, 1:2] != 0.0) | (q[:, 2:3] != 0.0)
    s = jnp.where(qvalid, s, 0.0)
    sum_ref[...] = jnp.sum(s).reshape(1, 1, 1)
    cnt_ref[...] = jnp.sum(qvalid.astype(f32)).reshape(1, 1, 1)


def kernel(source_pc, target_pc):
    B, NQ, _ = source_pc.shape
    NR = target_pc.shape[1]
    nqb = NQ // _QBLK
    tt = jnp.transpose(target_pc, (0, 2, 1))
    sums, cnts = pl.pallas_call(
        _knn_kernel,
        grid=(B, nqb),
        in_specs=[
            pl.BlockSpec((1, _QBLK, 3), lambda b, i: (b, i, 0)),
            pl.BlockSpec((1, 3, NR), lambda b, i: (b, 0, 0)),
        ],
        out_specs=[
            pl.BlockSpec((1, 1, 1), lambda b, i: (b * nqb + i, 0, 0)),
            pl.BlockSpec((1, 1, 1), lambda b, i: (b * nqb + i, 0, 0)),
        ],
        out_shape=[
            jax.ShapeDtypeStruct((B * nqb, 1, 1), jnp.float32),
            jax.ShapeDtypeStruct((B * nqb, 1, 1), jnp.float32),
        ],
        compiler_params=pltpu.CompilerParams(
            dimension_semantics=("parallel", "parallel")),
    )(source_pc, tt)
    total = jnp.sum(sums.reshape(B, nqb), axis=1)
    cnt = jnp.sum(cnts.reshape(B, nqb), axis=1) * _K
    return jnp.mean(total / cnt)
